# baseline (speedup 1.0000x reference)
import functools

import jax
import jax.numpy as jnp
import numpy as np
from jax import lax
from jax.experimental import pallas as pl
from jax.experimental.pallas import tpu as pltpu
from jax.experimental.pallas import tpu_sc as plsc

N = 16384
T = 50
MARGIN = 1.0
NUM_TILES = 16
ROWS_PER_TILE = N // NUM_TILES
CHUNK_ROWS = 128
CHUNKS_PER_TILE = ROWS_PER_TILE // CHUNK_ROWS
GROUPS_PER_CHUNK = CHUNK_ROWS // 16
BITS_PER_CHUNK = CHUNK_ROWS * T

def _rotl(x, r):
    return ((x << np.uint32(r)) | (x >> np.uint32(32 - r))).astype(np.uint32)


def _threefry2x32(ks0, ks1, x0, x1):
    rot_a = (13, 15, 26, 6)
    rot_b = (17, 29, 16, 24)
    ks2 = np.uint32(ks0 ^ ks1 ^ np.uint32(0x1BD11BDA))
    x0 = (x0 + ks0).astype(np.uint32)
    x1 = (x1 + ks1).astype(np.uint32)

    def four_rounds(x0, x1, rots):
        for r in rots:
            x0 = (x0 + x1).astype(np.uint32)
            x1 = _rotl(x1, r)
            x1 = x1 ^ x0
        return x0, x1

    sched = [(ks1, ks2), (ks2, ks0), (ks0, ks1), (ks1, ks2), (ks2, ks0)]
    for i, (a, b) in enumerate(sched):
        x0, x1 = four_rounds(x0, x1, rot_a if i % 2 == 0 else rot_b)
        x0 = (x0 + a).astype(np.uint32)
        x1 = (x1 + b + np.uint32(i + 1)).astype(np.uint32)
    return x0, x1


def _np_random_bits(key2, size):
    o0, o1 = _threefry2x32(key2[0], key2[1], np.zeros(size, np.uint32),
                           np.arange(size, dtype=np.uint32))
    return o0 ^ o1


def _np_split(key2):
    o0, o1 = _threefry2x32(key2[0], key2[1], np.zeros(2, np.uint32),
                           np.arange(2, dtype=np.uint32))
    return np.stack([o0, o1], axis=1)


_TABLES = None


def _tables():
    global _TABLES
    if _TABLES is None:
        k1, k2 = _np_split(np.array([0, 42], dtype=np.uint32))
        hi = _np_random_bits(k1, N * T).reshape(N, T)
        lo = _np_random_bits(k2, N * T).reshape(N, T)

        def permute(b):
            return np.ascontiguousarray(
                b.reshape(N // 16, 16, T).transpose(0, 2, 1)
            ).reshape(-1).view(np.int32)

        harm = np.zeros(64, np.float32)
        harm[:T] = np.cumsum((1.0 / np.arange(1, T + 1)).astype(np.float32),
                             dtype=np.float32)
        _TABLES = (permute(hi), permute(lo), harm)
    return _TABLES


def _warp_body(scores_h, labels_h, hi_h, lo_h, harm_h, out_h,
               scores_v, labels_v, pos_buf, neg_buf, hi_v, lo_v, harm_v,
               stage_v, red_v, partials_s):
    wid = lax.axis_index("s")
    base = wid * ROWS_PER_TILE
    iota = lax.iota(jnp.int32, 16)

    pltpu.sync_copy(scores_h, scores_v)
    pltpu.sync_copy(labels_h, labels_v)
    pltpu.sync_copy(harm_h, harm_v)

    scope1 = jax.named_scope("phase1_compact")
    scope1.__enter__()
    def scan_body(v, carry):
        off_p, off_n = carry
        s = scores_v[pl.ds(v * 16, 16)]
        l = labels_v[pl.ds(v * 16, 16)]
        mpos = l == 1
        mpos_i = mpos.astype(jnp.int32)
        cums = plsc.cumsum(mpos_i)
        cp = jnp.max(cums)
        excl_p = cums - mpos_i
        excl_n = iota - excl_p
        idx_p = jnp.where(mpos, off_p + excl_p, N + iota)
        idx_n = jnp.where(mpos, N + iota, off_n + excl_n)
        plsc.store_scatter(pos_buf, [idx_p], s)
        plsc.store_scatter(neg_buf, [idx_n], s)
        return off_p + cp, off_n + (16 - cp)

    num_pos, num_neg = lax.fori_loop(
        0, N // 16, scan_body, (jnp.int32(0), jnp.int32(0)))

    scope1.__exit__(None, None, None)
    scope2 = jax.named_scope("phase2_sample")
    scope2.__enter__()
    span_u = jnp.maximum(num_neg, 1).astype(jnp.uint32)
    m1 = jnp.uint32(65536) % span_u
    mult_u = (m1 * m1) % span_u

    def group_body(g, acc, start):
        r0 = start + g * 16
        pos16 = pos_buf[pl.ds(r0, 16)]
        done0 = ((r0 + iota) >= num_pos).astype(jnp.int32)
        bitbase = g * (16 * T)

        def cond(st):
            t, done, tstar, fneg = st
            return (t < T) & (jnp.min(done) == 0)

        def tbody(st):
            t, done, tstar, fneg = st
            hu = plsc.bitcast(hi_v[pl.ds(bitbase + t * 16, 16)], jnp.uint32)
            lu = plsc.bitcast(lo_v[pl.ds(bitbase + t * 16, 16)], jnp.uint32)
            offs = ((hu % span_u) * mult_u + (lu % span_u)) % span_u
            idx = offs.astype(jnp.int32)
            vals = plsc.load_gather(neg_buf, [idx])
            viol = (vals + MARGIN) > pos16
            newly = viol & (done == 0)
            tstar = jnp.where(newly, t, tstar)
            fneg = jnp.where(newly, vals, fneg)
            done = jnp.where(viol, 1, done)
            return t + 1, done, tstar, fneg

        _, _, tstar, fneg = lax.while_loop(
            cond, tbody,
            (jnp.int32(0), done0, jnp.full((16,), -1, jnp.int32),
             jnp.zeros((16,), jnp.float32)))

        tsafe = jnp.maximum(tstar, 0)
        rank = jnp.maximum(1, T // (tsafe + 1))
        w16 = plsc.load_gather(harm_v, [rank - 1])
        hinge = jnp.maximum(MARGIN - (pos16 - fneg), 0.0)
        contrib = jnp.where(tstar >= 0, w16 * hinge, 0.0)
        return acc + jnp.sum(contrib)

    def chunk_body(c, acc):
        start = base + c * CHUNK_ROWS

        def do(acc):
            pltpu.sync_copy(hi_h.at[pl.ds(start * T, BITS_PER_CHUNK)], hi_v)
            pltpu.sync_copy(lo_h.at[pl.ds(start * T, BITS_PER_CHUNK)], lo_v)
            return lax.fori_loop(
                0, GROUPS_PER_CHUNK,
                lambda g, a: group_body(g, a, start), acc)

        return lax.cond(start < num_pos, do, lambda a: a, acc)

    acc = lax.fori_loop(0, CHUNKS_PER_TILE, chunk_body, jnp.float32(0.0))

    scope2.__exit__(None, None, None)
    scope3 = jax.named_scope("phase3_reduce")
    scope3.__enter__()
    stage_v[...] = jnp.zeros((16,), jnp.float32) + acc
    pltpu.sync_copy(stage_v, partials_s.at[pl.ds(wid * 16, 16)])
    plsc.subcore_barrier()

    @pl.when(wid == 0)
    def _():
        pltpu.sync_copy(partials_s, red_v)
        tot = jnp.zeros((16,), jnp.float32)
        for i in range(NUM_TILES):
            tot = tot + red_v[pl.ds(i * 16, 16)]
        empty = (num_pos == 0) | (num_neg == 0)
        denom = jnp.maximum(num_pos, 1).astype(jnp.float32)
        denom_vec = jnp.zeros((16,), jnp.float32) + denom
        final_vec = jnp.where(empty, jnp.zeros((16,), jnp.float32),
                              tot / denom_vec)
        stage_v[...] = final_vec
        pltpu.sync_copy(stage_v, out_h)

    scope3.__exit__(None, None, None)


_WARP = None


def _get_warp():
    global _WARP
    if _WARP is None:
        mesh = plsc.VectorSubcoreMesh(
            core_axis_name="c", subcore_axis_name="s",
            num_cores=1, num_subcores=16)
        _WARP = functools.partial(
            pl.kernel,
            out_type=jax.ShapeDtypeStruct((16,), jnp.float32),
            mesh=mesh,
            compiler_params=pltpu.CompilerParams(needs_layout_passes=False),
            scratch_types=[
                pltpu.VMEM((N,), jnp.float32),
                pltpu.VMEM((N,), jnp.int32),
                pltpu.VMEM((N + 16,), jnp.float32),
                pltpu.VMEM((N + 16,), jnp.float32),
                pltpu.VMEM((BITS_PER_CHUNK,), jnp.int32),
                pltpu.VMEM((BITS_PER_CHUNK,), jnp.int32),
                pltpu.VMEM((64,), jnp.float32),
                pltpu.VMEM((16,), jnp.float32),
                pltpu.VMEM((16 * NUM_TILES,), jnp.float32),
                pltpu.VMEM_SHARED((16 * NUM_TILES,), jnp.float32),
            ],
        )(_warp_body)
    return _WARP


def kernel(scores, labels):
    hi_np, lo_np, harm_np = _tables()
    out = _get_warp()(scores, labels.astype(jnp.int32),
                      jnp.asarray(hi_np), jnp.asarray(lo_np),
                      jnp.asarray(harm_np))
    return out[0]


# speedup vs baseline: 1.0607x; 1.0607x over previous
import functools

import jax
import jax.numpy as jnp
import numpy as np
from jax import lax
from jax.experimental import pallas as pl
from jax.experimental.pallas import tpu as pltpu
from jax.experimental.pallas import tpu_sc as plsc

N = 16384
T = 50
MARGIN = 1.0
NUM_TILES = 16
ROWS_PER_TILE = N // NUM_TILES
CHUNK_ROWS = 128
CHUNKS_PER_TILE = ROWS_PER_TILE // CHUNK_ROWS
GROUPS_PER_CHUNK = CHUNK_ROWS // 16
BITS_PER_CHUNK = CHUNK_ROWS * T
TRIAL_BLOCK = 8
BITS_BUF = BITS_PER_CHUNK + 16 * TRIAL_BLOCK

def _rotl(x, r):
    return ((x << np.uint32(r)) | (x >> np.uint32(32 - r))).astype(np.uint32)


def _threefry2x32(ks0, ks1, x0, x1):
    rot_a = (13, 15, 26, 6)
    rot_b = (17, 29, 16, 24)
    ks2 = np.uint32(ks0 ^ ks1 ^ np.uint32(0x1BD11BDA))
    x0 = (x0 + ks0).astype(np.uint32)
    x1 = (x1 + ks1).astype(np.uint32)

    def four_rounds(x0, x1, rots):
        for r in rots:
            x0 = (x0 + x1).astype(np.uint32)
            x1 = _rotl(x1, r)
            x1 = x1 ^ x0
        return x0, x1

    sched = [(ks1, ks2), (ks2, ks0), (ks0, ks1), (ks1, ks2), (ks2, ks0)]
    for i, (a, b) in enumerate(sched):
        x0, x1 = four_rounds(x0, x1, rot_a if i % 2 == 0 else rot_b)
        x0 = (x0 + a).astype(np.uint32)
        x1 = (x1 + b + np.uint32(i + 1)).astype(np.uint32)
    return x0, x1


def _np_random_bits(key2, size):
    o0, o1 = _threefry2x32(key2[0], key2[1], np.zeros(size, np.uint32),
                           np.arange(size, dtype=np.uint32))
    return o0 ^ o1


def _np_split(key2):
    o0, o1 = _threefry2x32(key2[0], key2[1], np.zeros(2, np.uint32),
                           np.arange(2, dtype=np.uint32))
    return np.stack([o0, o1], axis=1)


_TABLES = None


def _tables():
    global _TABLES
    if _TABLES is None:
        k1, k2 = _np_split(np.array([0, 42], dtype=np.uint32))
        hi = _np_random_bits(k1, N * T).reshape(N, T)
        lo = _np_random_bits(k2, N * T).reshape(N, T)

        def permute(b):
            return np.ascontiguousarray(
                b.reshape(N // 16, 16, T).transpose(0, 2, 1)
            ).reshape(-1).view(np.int32)

        harm = np.zeros(64, np.float32)
        harm[:T] = np.cumsum((1.0 / np.arange(1, T + 1)).astype(np.float32),
                             dtype=np.float32)
        _TABLES = (permute(hi), permute(lo), harm)
    return _TABLES


def _warp_body(scores_h, labels_h, hi_h, lo_h, harm_h, out_h,
               scores_v, labels_v, pos_buf, neg_buf, hi_v, lo_v, harm_v,
               stage_v, red_v, partials_s):
    wid = lax.axis_index("s")
    base = wid * ROWS_PER_TILE
    iota = lax.iota(jnp.int32, 16)

    pltpu.sync_copy(scores_h, scores_v)
    pltpu.sync_copy(labels_h, labels_v)
    pltpu.sync_copy(harm_h, harm_v)

    def scan_body(v, carry):
        off_p, off_n = carry
        s = scores_v[pl.ds(v * 16, 16)]
        l = labels_v[pl.ds(v * 16, 16)]
        mpos = l == 1
        mpos_i = jnp.where(mpos, jnp.full((16,), 1, jnp.int32),
                           jnp.zeros((16,), jnp.int32))
        cums = plsc.cumsum(mpos_i)
        cp = plsc.all_reduce_population_count(mpos)
        excl_p = cums - mpos_i
        excl_n = iota - excl_p
        idx_p = jnp.where(mpos, off_p + excl_p, N + iota)
        idx_n = jnp.where(mpos, N + iota, off_n + excl_n)
        plsc.store_scatter(pos_buf, [idx_p], s)
        plsc.store_scatter(neg_buf, [idx_n], s)
        return off_p + cp, off_n + (16 - cp)

    num_pos_v, num_neg_v = lax.fori_loop(
        0, N // 16, scan_body,
        (jnp.zeros((16,), jnp.int32), jnp.zeros((16,), jnp.int32)))
    num_pos = jnp.max(num_pos_v)
    num_neg = jnp.max(num_neg_v)

    span_u = jnp.maximum(num_neg, 1).astype(jnp.uint32)
    m1 = jnp.uint32(65536) % span_u
    mult_u = (m1 * m1) % span_u

    ones_i = jnp.full((16,), 1, jnp.int32)

    def group_body(g, acc_vec, start):
        r0 = start + g * 16
        pos16 = pos_buf[pl.ds(r0, 16)]
        done0 = jnp.where((r0 + iota) >= num_pos, ones_i,
                          jnp.zeros((16,), jnp.int32))
        bitbase = g * (16 * T)

        def cond(st):
            tb, done, tstar, fneg = st
            return (tb < T) & (jnp.min(done) == 0)

        def bbody(st):
            tb, done, tstar, fneg = st
            for b in range(TRIAL_BLOCK):
                t = tb + b
                hu = plsc.bitcast(hi_v[pl.ds(bitbase + t * 16, 16)],
                                  jnp.uint32)
                lu = plsc.bitcast(lo_v[pl.ds(bitbase + t * 16, 16)],
                                  jnp.uint32)
                offs = ((hu % span_u) * mult_u + (lu % span_u)) % span_u
                idx = offs.astype(jnp.int32)
                vals = plsc.load_gather(neg_buf, [idx])
                viol = (vals + MARGIN) > pos16
                newly = viol & (done == 0) & (t < T)
                tstar = jnp.where(newly, t, tstar)
                fneg = jnp.where(newly, vals, fneg)
                done = jnp.where(viol, ones_i, done)
            return tb + TRIAL_BLOCK, done, tstar, fneg

        _, _, tstar, fneg = lax.while_loop(
            cond, bbody,
            (jnp.int32(0), done0, jnp.full((16,), -1, jnp.int32),
             jnp.zeros((16,), jnp.float32)))

        tsafe = jnp.maximum(tstar, 0)
        rank = jnp.maximum(1, T // (tsafe + 1))
        w16 = plsc.load_gather(harm_v, [rank - 1])
        hinge = jnp.maximum(MARGIN - (pos16 - fneg), 0.0)
        contrib = jnp.where(tstar >= 0, w16 * hinge, 0.0)
        return acc_vec + contrib

    def chunk_body(c, acc_vec):
        start = base + c * CHUNK_ROWS

        def do(acc_vec):
            pltpu.sync_copy(hi_h.at[pl.ds(start * T, BITS_PER_CHUNK)],
                            hi_v.at[pl.ds(0, BITS_PER_CHUNK)])
            pltpu.sync_copy(lo_h.at[pl.ds(start * T, BITS_PER_CHUNK)],
                            lo_v.at[pl.ds(0, BITS_PER_CHUNK)])
            return lax.fori_loop(
                0, GROUPS_PER_CHUNK,
                lambda g, a: group_body(g, a, start), acc_vec)

        return lax.cond(start < num_pos, do, lambda a: a, acc_vec)

    acc_vec = lax.fori_loop(0, CHUNKS_PER_TILE, chunk_body,
                            jnp.zeros((16,), jnp.float32))

    stage_v[...] = acc_vec
    pltpu.sync_copy(stage_v, partials_s.at[pl.ds(wid * 16, 16)])
    plsc.subcore_barrier()

    @pl.when(wid == 0)
    def _():
        pltpu.sync_copy(partials_s, red_v)
        tot = jnp.zeros((16,), jnp.float32)
        for i in range(NUM_TILES):
            tot = tot + red_v[pl.ds(i * 16, 16)]
        total = jnp.sum(tot)
        empty = (num_pos == 0) | (num_neg == 0)
        denom = jnp.maximum(num_pos, 1).astype(jnp.float32)
        denom_vec = jnp.zeros((16,), jnp.float32) + denom
        final_vec = jnp.where(empty, jnp.zeros((16,), jnp.float32),
                              (jnp.zeros((16,), jnp.float32) + total)
                              / denom_vec)
        stage_v[...] = final_vec
        pltpu.sync_copy(stage_v, out_h)


_WARP = None


def _get_warp():
    global _WARP
    if _WARP is None:
        mesh = plsc.VectorSubcoreMesh(
            core_axis_name="c", subcore_axis_name="s",
            num_cores=1, num_subcores=16)
        _WARP = functools.partial(
            pl.kernel,
            out_type=jax.ShapeDtypeStruct((16,), jnp.float32),
            mesh=mesh,
            compiler_params=pltpu.CompilerParams(needs_layout_passes=False),
            scratch_types=[
                pltpu.VMEM((N,), jnp.float32),
                pltpu.VMEM((N,), jnp.int32),
                pltpu.VMEM((N + 16,), jnp.float32),
                pltpu.VMEM((N + 16,), jnp.float32),
                pltpu.VMEM((BITS_BUF,), jnp.int32),
                pltpu.VMEM((BITS_BUF,), jnp.int32),
                pltpu.VMEM((64,), jnp.float32),
                pltpu.VMEM((16,), jnp.float32),
                pltpu.VMEM((16 * NUM_TILES,), jnp.float32),
                pltpu.VMEM_SHARED((16 * NUM_TILES,), jnp.float32),
            ],
        )(_warp_body)
    return _WARP


def kernel(scores, labels):
    hi_np, lo_np, harm_np = _tables()
    out = _get_warp()(scores, labels.astype(jnp.int32),
                      jnp.asarray(hi_np), jnp.asarray(lo_np),
                      jnp.asarray(harm_np))
    return out[0]
